# baseline (speedup 1.0000x reference)
import jax
import jax.numpy as jnp
from jax.experimental import pallas as pl
from jax.experimental.pallas import tpu as pltpu

DIM = 384
HEADS = 8
HD = DIM // HEADS
H = 224
W = 224
NPIX = H * W
TP = 1792
PAD = 256
SLAB = TP + 2 * PAD
NSTEP = NPIX // TP
TOPK = 7

_F32 = jnp.float32
_BF16 = jnp.bfloat16


def _mm_nn(a, b):
    return jax.lax.dot_general(a, b, (((1,), (0,)), ((), ())),
                               preferred_element_type=_F32)


def _mm_tn(a, b):
    return jax.lax.dot_general(a, b, (((0,), (0,)), ((), ())),
                               preferred_element_type=_F32)


def _mm_nt(a, b):
    return jax.lax.dot_general(a, b, (((1,), (1,)), ((), ())),
                               preferred_element_type=_F32)


def _mega(x_hbm, wqkv, wdw, trow, wp,
          q_hbm, k_hbm, v_hbm, out_ref,
          xbuf, xsem, st_q, st_k, st_v, wsem,
          qin, kin, isem, vin, vsem,
          aux_s, rr_s, g_s, m_s):
    i = pl.program_id(0)

    def cp_first(slot):
        return pltpu.make_async_copy(
            x_hbm.at[pl.ds(0, TP + PAD), :],
            xbuf.at[slot, pl.ds(PAD, TP + PAD), :], xsem.at[slot])

    def cp_mid(slot, idx):
        return pltpu.make_async_copy(
            x_hbm.at[pl.ds(idx * TP - PAD, SLAB), :],
            xbuf.at[slot], xsem.at[slot])

    def cp_last(slot):
        return pltpu.make_async_copy(
            x_hbm.at[pl.ds((NSTEP - 1) * TP - PAD, TP + PAD), :],
            xbuf.at[slot, pl.ds(0, TP + PAD), :], xsem.at[slot])

    def start_x(slot, idx):
        @pl.when(idx < NSTEP - 1)
        def _():
            cp_mid(slot, idx).start()

        @pl.when(idx == NSTEP - 1)
        def _():
            xbuf[slot, TP + PAD:SLAB, :] = jnp.zeros((PAD, DIM), _F32)
            cp_last(slot).start()

    def wait_x(slot, idx):
        @pl.when(idx == 0)
        def _():
            cp_first(slot).wait()

        @pl.when(jnp.logical_and(idx > 0, idx < NSTEP - 1))
        def _():
            cp_mid(slot, idx).wait()

        @pl.when(idx == NSTEP - 1)
        def _():
            cp_last(slot).wait()

    def cpw(st, hbm, idx, s):
        return pltpu.make_async_copy(st, hbm.at[pl.ds(idx * TP, TP), :],
                                     wsem.at[s])

    def cpi_q(slot, j):
        return pltpu.make_async_copy(q_hbm.at[pl.ds(j * TP, TP), :],
                                     qin.at[slot], isem.at[slot, 0])

    def cpi_k(slot, j):
        return pltpu.make_async_copy(k_hbm.at[pl.ds(j * TP, TP), :],
                                     kin.at[slot], isem.at[slot, 1])

    def cpv(slot, j):
        return pltpu.make_async_copy(v_hbm.at[pl.ds(j * TP, TP), :],
                                     vin.at[slot], vsem.at[slot])

    @pl.when(i == 0)
    def _():
        xbuf[0, 0:PAD, :] = jnp.zeros((PAD, DIM), _F32)
        cp_first(0).start()

    @pl.when(jnp.logical_and(i + 1 < NSTEP, i < NSTEP))
    def _():
        start_x((i + 1) % 2, i + 1)

    @pl.when(i < NSTEP)
    def _phase_a():
        slot = i % 2
        wait_x(slot, i)
        xs = xbuf[slot].astype(_BF16)

        col = jax.lax.broadcasted_iota(jnp.int32, (TP, 1), 0) % W
        m_l = (col > 0).astype(_F32)
        m_r = (col < W - 1).astype(_F32)

        def dwconv(raw_b, w72):
            c = raw_b.shape[1]
            raw3 = raw_b.astype(_F32).reshape(SLAB // 8, 8, c)
            w3 = w72.reshape(9, 8, c)

            def vert(dc_idx, start, n):
                s0 = (start - W) // 8
                s1 = start // 8
                s2 = (start + W) // 8
                n8 = n // 8
                return (w3[dc_idx:dc_idx + 1] * raw3[s0:s0 + n8]
                        + w3[dc_idx + 3:dc_idx + 4] * raw3[s1:s1 + n8]
                        + w3[dc_idx + 6:dc_idx + 7] * raw3[s2:s2 + n8])

            vc = vert(1, PAD, TP).reshape(TP, c)
            vl = vert(0, PAD - 8, TP + 8).reshape(TP + 8, c)
            vr = vert(2, PAD, TP + 8).reshape(TP + 8, c)
            return vc + m_l * vl[7:7 + TP, :] + m_r * vr[1:1 + TP, :]

        q = dwconv(_mm_nt(xs, wqkv[0:DIM, :]).astype(_BF16), wdw[:, 0:DIM])
        k = dwconv(_mm_nt(xs, wqkv[DIM:2 * DIM, :]).astype(_BF16),
                   wdw[:, DIM:2 * DIM])
        v = dwconv(_mm_nt(xs, wqkv[2 * DIM:3 * DIM, :]).astype(_BF16),
                   wdw[:, 2 * DIM:3 * DIM])

        @pl.when(i >= 1)
        def _():
            cpw(st_q, q_hbm, i - 1, 0).wait()
            cpw(st_k, k_hbm, i - 1, 1).wait()
            cpw(st_v, v_hbm, i - 1, 2).wait()

        st_q[...] = q
        st_k[...] = k
        st_v[...] = v.astype(_BF16)
        cpw(st_q, q_hbm, i, 0).start()
        cpw(st_k, k_hbm, i, 1).start()
        cpw(st_v, v_hbm, i, 2).start()

        qn2 = jnp.sum(q * q, axis=0, keepdims=True)
        kn2 = jnp.sum(k * k, axis=0, keepdims=True)
        nrm = jnp.concatenate([qn2, kn2, qn2, kn2, qn2, kn2, qn2, kn2],
                              axis=0)

        @pl.when(i == 0)
        def _():
            aux_s[...] = nrm

        @pl.when(i > 0)
        def _():
            aux_s[...] += nrm

    @pl.when(i == NSTEP)
    def _():
        cpw(st_q, q_hbm, NSTEP - 1, 0).wait()
        cpw(st_k, k_hbm, NSTEP - 1, 1).wait()
        cpw(st_v, v_hbm, NSTEP - 1, 2).wait()
        rr_s[...] = 1.0 / jnp.maximum(jnp.sqrt(aux_s[...]), 1e-12)
        cpi_q(0, 0).start()
        cpi_k(0, 0).start()

    @pl.when(jnp.logical_and(i >= NSTEP, i < 2 * NSTEP))
    def _phase_g():
        j = i - NSTEP
        slot = j % 2

        @pl.when(j + 1 < NSTEP)
        def _():
            cpi_q((j + 1) % 2, j + 1).start()
            cpi_k((j + 1) % 2, j + 1).start()

        cpi_q(slot, j).wait()
        cpi_k(slot, j).wait()
        qn = (qin[slot] * rr_s[0:1, :]).astype(_BF16)
        kn = (kin[slot] * rr_s[1:2, :]).astype(_BF16)
        g = _mm_tn(kn, qn)

        @pl.when(j == 0)
        def _():
            g_s[...] = g

        @pl.when(j > 0)
        def _():
            g_s[...] += g

    @pl.when(i == 2 * NSTEP)
    def _():
        s = g_s[...] * trow[...]
        rows = jax.lax.broadcasted_iota(jnp.int32, (DIM, DIM), 0) // HD
        cols = jax.lax.broadcasted_iota(jnp.int32, (DIM, DIM), 1) // HD
        neg = _F32(-jnp.inf)
        s = jnp.where(rows == cols, s, neg)
        work = s
        m0 = thr = None
        for t in range(TOPK):
            thr = jnp.max(work, axis=0, keepdims=True)
            if t == 0:
                m0 = thr
            work = jnp.where(work == thr, neg, work)
        p = jnp.where(s >= thr, jnp.exp(s - m0), 0.0)
        b = p / jnp.sum(p, axis=0, keepdims=True)
        m_s[...] = _mm_nt(b.astype(_BF16), wp[...]).astype(_BF16)
        cpv(0, 0).start()

    @pl.when(i >= 2 * NSTEP)
    def _phase_o():
        j = i - 2 * NSTEP
        slot = j % 2

        @pl.when(j + 1 < NSTEP)
        def _():
            cpv((j + 1) % 2, j + 1).start()

        cpv(slot, j).wait()
        out_ref[...] = _mm_nn(vin[slot], m_s[...])


def kernel(x, W_qkv, W_dw, W_proj, temperature):
    xt = jnp.transpose(x.reshape(DIM, NPIX))

    wqkv = W_qkv.reshape(3 * DIM, DIM).astype(_BF16)
    wdw9 = jnp.transpose(W_dw.reshape(3 * DIM, 9))
    wdw = jnp.broadcast_to(wdw9[:, None, :], (9, 8, 3 * DIM)).reshape(72, 3 * DIM)
    trow = jnp.repeat(temperature.reshape(HEADS), HD).reshape(1, DIM)
    wp = W_proj.reshape(DIM, DIM).astype(_BF16)

    _, _, _, out = pl.pallas_call(
        _mega,
        grid=(3 * NSTEP,),
        in_specs=[
            pl.BlockSpec(memory_space=pl.ANY),
            pl.BlockSpec((3 * DIM, DIM), lambda i: (0, 0)),
            pl.BlockSpec((72, 3 * DIM), lambda i: (0, 0)),
            pl.BlockSpec((1, DIM), lambda i: (0, 0)),
            pl.BlockSpec((DIM, DIM), lambda i: (0, 0)),
        ],
        out_specs=[
            pl.BlockSpec(memory_space=pl.ANY),
            pl.BlockSpec(memory_space=pl.ANY),
            pl.BlockSpec(memory_space=pl.ANY),
            pl.BlockSpec((TP, DIM), lambda i: (jnp.maximum(i - 2 * NSTEP, 0), 0)),
        ],
        out_shape=[
            jax.ShapeDtypeStruct((NPIX, DIM), _F32),
            jax.ShapeDtypeStruct((NPIX, DIM), _F32),
            jax.ShapeDtypeStruct((NPIX, DIM), _BF16),
            jax.ShapeDtypeStruct((NPIX, DIM), _F32),
        ],
        scratch_shapes=[
            pltpu.VMEM((2, SLAB, DIM), _F32),
            pltpu.SemaphoreType.DMA((2,)),
            pltpu.VMEM((TP, DIM), _F32),
            pltpu.VMEM((TP, DIM), _F32),
            pltpu.VMEM((TP, DIM), _BF16),
            pltpu.SemaphoreType.DMA((3,)),
            pltpu.VMEM((2, TP, DIM), _F32),
            pltpu.VMEM((2, TP, DIM), _F32),
            pltpu.SemaphoreType.DMA((2, 2)),
            pltpu.VMEM((2, TP, DIM), _BF16),
            pltpu.SemaphoreType.DMA((2,)),
            pltpu.VMEM((8, DIM), _F32),
            pltpu.VMEM((8, DIM), _F32),
            pltpu.VMEM((DIM, DIM), _F32),
            pltpu.VMEM((DIM, DIM), _BF16),
        ],
        compiler_params=pltpu.CompilerParams(
            dimension_semantics=("arbitrary",),
        ),
    )(xt, wqkv, wdw, trow, wp)

    return jnp.transpose(out).reshape(1, DIM, H, W)


# speedup vs baseline: 1.0630x; 1.0630x over previous
import jax
import jax.numpy as jnp
from jax.experimental import pallas as pl
from jax.experimental.pallas import tpu as pltpu

DIM = 384
HEADS = 8
HD = DIM // HEADS
H = 224
W = 224
NPIX = H * W
TP = 1792
PAD = 256
SLAB = TP + 2 * PAD
NSTEP = NPIX // TP
TP2 = 3584
NSTEP2 = NPIX // TP2
TOPK = 7

_F32 = jnp.float32
_BF16 = jnp.bfloat16


def _mm_nn(a, b):
    return jax.lax.dot_general(a, b, (((1,), (0,)), ((), ())),
                               preferred_element_type=_F32)


def _mm_tn(a, b):
    return jax.lax.dot_general(a, b, (((0,), (0,)), ((), ())),
                               preferred_element_type=_F32)


def _mm_nt(a, b):
    return jax.lax.dot_general(a, b, (((1,), (1,)), ((), ())),
                               preferred_element_type=_F32)


def _pass_a(x_hbm, wqkv, wdw,
            q_ref, k_ref, v_ref, aux_ref,
            xbuf, sem):
    i = pl.program_id(0)

    def cp_first(slot):
        return pltpu.make_async_copy(
            x_hbm.at[pl.ds(0, TP + PAD), :],
            xbuf.at[slot, pl.ds(PAD, TP + PAD), :], sem.at[slot])

    def cp_mid(slot, idx):
        return pltpu.make_async_copy(
            x_hbm.at[pl.ds(idx * TP - PAD, SLAB), :],
            xbuf.at[slot], sem.at[slot])

    def cp_last(slot):
        return pltpu.make_async_copy(
            x_hbm.at[pl.ds((NSTEP - 1) * TP - PAD, TP + PAD), :],
            xbuf.at[slot, pl.ds(0, TP + PAD), :], sem.at[slot])

    def start_copy(slot, idx):
        @pl.when(idx < NSTEP - 1)
        def _():
            cp_mid(slot, idx).start()

        @pl.when(idx == NSTEP - 1)
        def _():
            xbuf[slot, TP + PAD:SLAB, :] = jnp.zeros((PAD, DIM), _F32)
            cp_last(slot).start()

    def wait_copy(slot, idx):
        @pl.when(idx == 0)
        def _():
            cp_first(slot).wait()

        @pl.when(jnp.logical_and(idx > 0, idx < NSTEP - 1))
        def _():
            cp_mid(slot, idx).wait()

        @pl.when(idx == NSTEP - 1)
        def _():
            cp_last(slot).wait()

    @pl.when(i == 0)
    def _():
        xbuf[0, 0:PAD, :] = jnp.zeros((PAD, DIM), _F32)
        cp_first(0).start()

    @pl.when(i + 1 < NSTEP)
    def _():
        start_copy((i + 1) % 2, i + 1)

    slot = i % 2
    wait_copy(slot, i)
    xs = xbuf[slot].astype(_BF16)

    col = jax.lax.broadcasted_iota(jnp.int32, (TP, 1), 0) % W
    m_l = (col > 0).astype(_F32)
    m_r = (col < W - 1).astype(_F32)

    def dwconv(raw_b, w72):
        c = raw_b.shape[1]
        raw3 = raw_b.astype(_F32).reshape(SLAB // 8, 8, c)
        w3 = w72.reshape(9, 8, c)

        def vert(dc_idx, start, n):
            s0 = (start - W) // 8
            s1 = start // 8
            s2 = (start + W) // 8
            n8 = n // 8
            return (w3[dc_idx:dc_idx + 1] * raw3[s0:s0 + n8]
                    + w3[dc_idx + 3:dc_idx + 4] * raw3[s1:s1 + n8]
                    + w3[dc_idx + 6:dc_idx + 7] * raw3[s2:s2 + n8])

        vc = vert(1, PAD, TP).reshape(TP, c)
        vl = vert(0, PAD - 8, TP + 8).reshape(TP + 8, c)
        vr = vert(2, PAD, TP + 8).reshape(TP + 8, c)
        return vc + m_l * vl[7:7 + TP, :] + m_r * vr[1:1 + TP, :]

    q = dwconv(_mm_nt(xs, wqkv[0:DIM, :]).astype(_BF16),
               wdw[:, 0:DIM])
    k = dwconv(_mm_nt(xs, wqkv[DIM:2 * DIM, :]).astype(_BF16),
               wdw[:, DIM:2 * DIM])
    q_ref[...] = q
    k_ref[...] = k

    v = dwconv(_mm_nt(xs, wqkv[2 * DIM:3 * DIM, :]).astype(_BF16),
               wdw[:, 2 * DIM:3 * DIM])
    v_ref[...] = v.astype(_BF16)

    qn2 = jnp.sum(q * q, axis=0, keepdims=True)
    kn2 = jnp.sum(k * k, axis=0, keepdims=True)
    nrm = jnp.concatenate([qn2, kn2, qn2, kn2, qn2, kn2, qn2, kn2], axis=0)

    @pl.when(i == 0)
    def _():
        aux_ref[...] = nrm

    @pl.when(i > 0)
    def _():
        aux_ref[...] += nrm


def _pass_bc(q_hbm, k_hbm, aux_ref, trow_ref, wp_ref, v_ref,
             out_ref, rr_ref, g_ref, m_ref):
    i = pl.program_id(0)

    @pl.when(i == 0)
    def _():
        rr_ref[...] = 1.0 / jnp.maximum(jnp.sqrt(aux_ref[...]), 1e-12)

    @pl.when(i < NSTEP2)
    def _():
        qn = (q_hbm[...] * rr_ref[0:1, :]).astype(_BF16)
        kn = (k_hbm[...] * rr_ref[1:2, :]).astype(_BF16)
        g = _mm_tn(kn, qn)

        @pl.when(i == 0)
        def _():
            g_ref[...] = g

        @pl.when(i > 0)
        def _():
            g_ref[...] += g

    @pl.when(i == NSTEP2)
    def _():
        s = g_ref[...] * trow_ref[...]
        rows = jax.lax.broadcasted_iota(jnp.int32, (DIM, DIM), 0) // HD
        cols = jax.lax.broadcasted_iota(jnp.int32, (DIM, DIM), 1) // HD
        neg = _F32(-jnp.inf)
        s = jnp.where(rows == cols, s, neg)
        work = s
        m0 = thr = None
        for t in range(TOPK):
            thr = jnp.max(work, axis=0, keepdims=True)
            if t == 0:
                m0 = thr
            work = jnp.where(work == thr, neg, work)
        p = jnp.where(s >= thr, jnp.exp(s - m0), 0.0)
        b = p / jnp.sum(p, axis=0, keepdims=True)
        m_mat = _mm_nt(b.astype(_BF16), wp_ref[...])
        m_ref[...] = m_mat.astype(_BF16)

    @pl.when(i >= NSTEP2)
    def _():
        out_ref[...] = _mm_nn(v_ref[...], m_ref[...])


def kernel(x, W_qkv, W_dw, W_proj, temperature):
    xt = jnp.transpose(x.reshape(DIM, NPIX))

    wqkv = W_qkv.reshape(3 * DIM, DIM).astype(_BF16)
    wdw9 = jnp.transpose(W_dw.reshape(3 * DIM, 9))
    wdw = jnp.broadcast_to(wdw9[:, None, :], (9, 8, 3 * DIM)).reshape(72, 3 * DIM)
    trow = jnp.repeat(temperature.reshape(HEADS), HD).reshape(1, DIM)
    wp = W_proj.reshape(DIM, DIM).astype(_BF16)

    q, k, v, aux = pl.pallas_call(
        _pass_a,
        grid=(NSTEP,),
        in_specs=[
            pl.BlockSpec(memory_space=pl.ANY),
            pl.BlockSpec((3 * DIM, DIM), lambda i: (0, 0)),
            pl.BlockSpec((72, 3 * DIM), lambda i: (0, 0)),
        ],
        out_specs=[
            pl.BlockSpec((TP, DIM), lambda i: (i, 0)),
            pl.BlockSpec((TP, DIM), lambda i: (i, 0)),
            pl.BlockSpec((TP, DIM), lambda i: (i, 0)),
            pl.BlockSpec((8, DIM), lambda i: (0, 0)),
        ],
        out_shape=[
            jax.ShapeDtypeStruct((NPIX, DIM), _F32),
            jax.ShapeDtypeStruct((NPIX, DIM), _F32),
            jax.ShapeDtypeStruct((NPIX, DIM), _BF16),
            jax.ShapeDtypeStruct((8, DIM), _F32),
        ],
        scratch_shapes=[
            pltpu.VMEM((2, SLAB, DIM), _F32),
            pltpu.SemaphoreType.DMA((2,)),
        ],
        compiler_params=pltpu.CompilerParams(
            dimension_semantics=("arbitrary",),
        ),
    )(xt, wqkv, wdw)

    out = pl.pallas_call(
        _pass_bc,
        grid=(2 * NSTEP2,),
        in_specs=[
            pl.BlockSpec((TP2, DIM), lambda i: (jnp.minimum(i, NSTEP2 - 1), 0)),
            pl.BlockSpec((TP2, DIM), lambda i: (jnp.minimum(i, NSTEP2 - 1), 0)),
            pl.BlockSpec((8, DIM), lambda i: (0, 0)),
            pl.BlockSpec((1, DIM), lambda i: (0, 0)),
            pl.BlockSpec((DIM, DIM), lambda i: (0, 0)),
            pl.BlockSpec((TP2, DIM), lambda i: (jnp.maximum(i - NSTEP2, 0), 0)),
        ],
        out_specs=pl.BlockSpec((TP2, DIM), lambda i: (jnp.maximum(i - NSTEP2, 0), 0)),
        out_shape=jax.ShapeDtypeStruct((NPIX, DIM), _F32),
        scratch_shapes=[
            pltpu.VMEM((8, DIM), _F32),
            pltpu.VMEM((DIM, DIM), _F32),
            pltpu.VMEM((DIM, DIM), _BF16),
        ],
        compiler_params=pltpu.CompilerParams(
            dimension_semantics=("arbitrary",),
        ),
    )(q, k, aux, trow, wp, v)

    return jnp.transpose(out).reshape(1, DIM, H, W)
